# baseline (speedup 1.0000x reference)
import jax
import jax.numpy as jnp
from jax import lax
from jax.experimental import pallas as pl
from jax.experimental.pallas import tpu as pltpu
from jax.experimental.pallas import tpu_sc as plsc

NC, NS, L = 2, 16, 16
NW = NC * NS
R, D = 4096, 8192
ROWS_PER_W = R // NW
VECS = D // L
C = 2
CHUNKS_PER_W = ROWS_PER_W // C
NBUF = 2


def _gather_body(attr_hbm, idx_hbm, out_hbm, *refs):
    abuf = refs[0:NBUF]
    ibuf = refs[NBUF:2 * NBUF]
    obuf = refs[2 * NBUF:3 * NBUF]
    sem_in = refs[3 * NBUF:4 * NBUF]
    sem_out = refs[4 * NBUF:5 * NBUF]

    wid = lax.axis_index("s") * NC + lax.axis_index("c")
    base = wid * ROWS_PER_W

    for b in range(NBUF - 1):
        pltpu.async_copy(attr_hbm.at[pl.ds(base + b * C, C)], abuf[b],
                         sem_in[b])
        pltpu.async_copy(idx_hbm.at[pl.ds(base + b * C, C)], ibuf[b],
                         sem_in[b])

    def block(q0, carry):
        for b in range(NBUF):
            q = q0 * NBUF + b
            row = base + q * C

            @pl.when(q + NBUF - 1 < CHUNKS_PER_W)
            def _():
                nb = (b + NBUF - 1) % NBUF
                nrow = row + (NBUF - 1) * C
                pltpu.async_copy(attr_hbm.at[pl.ds(nrow, C)], abuf[nb],
                                 sem_in[nb])
                pltpu.async_copy(idx_hbm.at[pl.ds(nrow, C)], ibuf[nb],
                                 sem_in[nb])

            pltpu.make_async_copy(attr_hbm.at[pl.ds(row, C)], abuf[b],
                                  sem_in[b]).wait()
            pltpu.make_async_copy(idx_hbm.at[pl.ds(row, C)], ibuf[b],
                                  sem_in[b]).wait()

            @pl.when(q >= NBUF)
            def _():
                pltpu.make_async_copy(obuf[b],
                                      out_hbm.at[pl.ds(row - NBUF * C, C)],
                                      sem_out[b]).wait()

            ab, ib, ob = abuf[b], ibuf[b], obuf[b]
            for c in range(C):
                cvec = jnp.full((L,), c, jnp.int32)

                @plsc.parallel_loop(0, VECS, unroll=8)
                def _(i):
                    idx = ib[c, pl.ds(i * L, L)]
                    ob[c, pl.ds(i * L, L)] = plsc.load_gather(
                        ab, [cvec, idx])

            pltpu.async_copy(ob, out_hbm.at[pl.ds(row, C)], sem_out[b])
        return carry

    lax.fori_loop(0, CHUNKS_PER_W // NBUF, block, 0)

    for b in range(NBUF):
        row = base + (CHUNKS_PER_W - NBUF + b) * C
        pltpu.make_async_copy(obuf[b], out_hbm.at[pl.ds(row, C)],
                              sem_out[b]).wait()


@jax.jit
def kernel(attr, batch_idxs):
    mesh = plsc.VectorSubcoreMesh(
        core_axis_name="c", subcore_axis_name="s", num_cores=NC, num_subcores=NS
    )
    k = pl.kernel(
        _gather_body,
        out_type=jax.ShapeDtypeStruct((R, D), jnp.float32),
        mesh=mesh,
        scratch_types=(
            [pltpu.VMEM((C, D), jnp.float32) for _ in range(NBUF)]
            + [pltpu.VMEM((C, D), jnp.int32) for _ in range(NBUF)]
            + [pltpu.VMEM((C, D), jnp.float32) for _ in range(NBUF)]
            + [pltpu.SemaphoreType.DMA for _ in range(2 * NBUF)]
        ),
        compiler_params=pltpu.CompilerParams(needs_layout_passes=False),
    )
    return k(attr, batch_idxs)


# speedup vs baseline: 1.0346x; 1.0346x over previous
import jax
import jax.numpy as jnp
from jax import lax
from jax.experimental import pallas as pl
from jax.experimental.pallas import tpu as pltpu
from jax.experimental.pallas import tpu_sc as plsc

NC, NS, L = 2, 16, 16
NW = NC * NS
R, D = 4096, 8192
ROWS_PER_W = R // NW
VECS = D // L
NBUF = 4


def _gather_body(attr_hbm, idx_hbm, out_hbm, *refs):
    arow = refs[0:NBUF]
    irow = refs[NBUF:2 * NBUF]
    orow = refs[2 * NBUF:3 * NBUF]
    sem_in = refs[3 * NBUF:4 * NBUF]
    sem_out = refs[4 * NBUF:5 * NBUF]

    wid = lax.axis_index("s") * NC + lax.axis_index("c")
    base = wid * ROWS_PER_W

    for b in range(NBUF - 1):
        pltpu.async_copy(attr_hbm.at[base + b], arow[b], sem_in[b])
        pltpu.async_copy(idx_hbm.at[base + b], irow[b], sem_in[b])

    def block(r0, carry):
        for b in range(NBUF):
            r = r0 * NBUF + b
            row = base + r

            @pl.when(r + NBUF - 1 < ROWS_PER_W)
            def _():
                nb = (b + NBUF - 1) % NBUF
                pltpu.async_copy(attr_hbm.at[row + NBUF - 1], arow[nb],
                                 sem_in[nb])
                pltpu.async_copy(idx_hbm.at[row + NBUF - 1], irow[nb],
                                 sem_in[nb])

            pltpu.make_async_copy(attr_hbm.at[row], arow[b],
                                  sem_in[b]).wait()
            pltpu.make_async_copy(idx_hbm.at[row], irow[b],
                                  sem_in[b]).wait()

            @pl.when(r >= NBUF)
            def _():
                pltpu.make_async_copy(orow[b], out_hbm.at[row - NBUF],
                                      sem_out[b]).wait()

            ab, ib, ob = arow[b], irow[b], orow[b]

            @plsc.parallel_loop(0, VECS, unroll=8)
            def _(i):
                idx = ib[pl.ds(i * L, L)]
                ob[pl.ds(i * L, L)] = plsc.load_gather(ab, [idx])

            pltpu.async_copy(ob, out_hbm.at[row], sem_out[b])
        return carry

    lax.fori_loop(0, ROWS_PER_W // NBUF, block, 0)

    for b in range(NBUF):
        pltpu.make_async_copy(orow[b],
                              out_hbm.at[base + ROWS_PER_W - NBUF + b],
                              sem_out[b]).wait()


@jax.jit
def kernel(attr, batch_idxs):
    mesh = plsc.VectorSubcoreMesh(
        core_axis_name="c", subcore_axis_name="s", num_cores=NC, num_subcores=NS
    )
    k = pl.kernel(
        _gather_body,
        out_type=jax.ShapeDtypeStruct((R, D), jnp.float32),
        mesh=mesh,
        scratch_types=(
            [pltpu.VMEM((D,), jnp.float32) for _ in range(NBUF)]
            + [pltpu.VMEM((D,), jnp.int32) for _ in range(NBUF)]
            + [pltpu.VMEM((D,), jnp.float32) for _ in range(NBUF)]
            + [pltpu.SemaphoreType.DMA for _ in range(2 * NBUF)]
        ),
        compiler_params=pltpu.CompilerParams(needs_layout_passes=False),
    )
    return k(attr, batch_idxs)
